# baseline (speedup 1.0000x reference)
import functools
import math

import jax
import jax.numpy as jnp
from jax import lax
from jax.experimental import pallas as pl
from jax.experimental.pallas import tpu as pltpu
from jax.experimental.pallas import tpu_sc as plsc

D_MODEL = 64
SEQ = 50
SCALE = math.sqrt(D_MODEL)

_info = plsc.get_sparse_core_info()
NC = _info.num_cores
NS = _info.num_subcores
NW = NC * NS

CHUNK_BATCH = 4
CHUNK_ROWS = CHUNK_BATCH * SEQ
SUBS = ((0, 128), (128, 72))
TBLK = 8192


def _widen_transpose(table_t, vocab):
    grid = (vocab + TBLK - 1) // TBLK

    def tbody(x_ref, o_ref):
        x = x_ref[...]
        eye = (
            lax.broadcasted_iota(jnp.int32, (D_MODEL, D_MODEL), 0)
            == lax.broadcasted_iota(jnp.int32, (D_MODEL, D_MODEL), 1)
        ).astype(jnp.float32)
        o_ref[:, 0:D_MODEL] = lax.dot_general(
            x, eye, (((0,), (0,)), ((), ())),
            precision=lax.Precision.HIGHEST,
        )

    return pl.pallas_call(
        tbody,
        grid=(grid,),
        in_specs=[pl.BlockSpec((D_MODEL, TBLK), lambda i: (0, i))],
        out_specs=pl.BlockSpec((TBLK, 128), lambda i: (i, 0)),
        out_shape=jax.ShapeDtypeStruct((vocab, 128), jnp.float32),
    )(table_t)


def _gather_pe(idx_flat, wide, pe_flat, batch, seq):
    n_rows = batch * seq
    rows_per_w = n_rows // NW
    n_chunks = rows_per_w // CHUNK_ROWS
    batches_per_w = rows_per_w // seq
    mesh = plsc.VectorSubcoreMesh(core_axis_name="c", subcore_axis_name="s")

    @functools.partial(
        pl.kernel,
        mesh=mesh,
        out_type=jax.ShapeDtypeStruct((batch, seq, D_MODEL), jnp.float32),
        scratch_types=[
            pltpu.VMEM((rows_per_w,), jnp.int32),
            [pltpu.VMEM((CHUNK_ROWS, 128), jnp.float32)] * 2,
            [pltpu.VMEM((CHUNK_BATCH, SEQ, D_MODEL), jnp.float32)] * 2,
            pltpu.VMEM((SEQ * D_MODEL,), jnp.float32),
            [pltpu.SemaphoreType.DMA] * 2,
            [pltpu.SemaphoreType.DMA] * 2,
        ],
        compiler_params=pltpu.CompilerParams(use_tc_tiling_on_sc=True),
    )
    def body(idx_hbm, wide_hbm, pe_hbm, out_hbm, idx_all, gbufs, obufs,
             pe_v, gsems, wsems):
        wid = lax.axis_index("s") * NC + lax.axis_index("c")
        wbase = pl.multiple_of(wid * rows_per_w, rows_per_w)
        wbatch = wid * batches_per_w
        pltpu.sync_copy(pe_hbm, pe_v)
        pltpu.sync_copy(idx_hbm.at[pl.ds(wbase, rows_per_w)], idx_all)

        def start_gather(c, gb):
            for off, sz in SUBS:
                pltpu.async_copy(
                    wide_hbm.at[
                        idx_all.at[
                            pl.ds(pl.multiple_of(c * CHUNK_ROWS + off, 8), sz)
                        ]
                    ],
                    gbufs[gb].at[pl.ds(off, sz)],
                    gsems[gb],
                )

        def wait_gather(gb):
            for off, sz in SUBS:
                pltpu.make_async_copy(
                    wide_hbm.at[idx_all.at[pl.ds(off, sz)]],
                    gbufs[gb].at[pl.ds(off, sz)],
                    gsems[gb],
                ).wait()

        def start_wb(c, ob):
            for bb in range(CHUNK_BATCH):
                pltpu.async_copy(
                    obufs[ob].at[bb],
                    out_hbm.at[wbatch + c * CHUNK_BATCH + bb],
                    wsems[ob],
                )

        def wait_wb(ob):
            for bb in range(CHUNK_BATCH):
                pltpu.make_async_copy(
                    obufs[ob].at[bb],
                    out_hbm.at[wbatch],
                    wsems[ob],
                ).wait()

        def compute(gb, ob):
            def pos_body(s, carry):
                for q in range(D_MODEL // 16):
                    peq = pe_v[pl.ds(pl.multiple_of(s * D_MODEL, 8) + q * 16,
                                     16)]
                    for bb in range(CHUNK_BATCH):
                        obufs[ob][bb, s, pl.ds(q * 16, 16)] = (
                            gbufs[gb][bb * SEQ + s, pl.ds(q * 16, 16)] * SCALE
                            + peq
                        )
                return carry

            lax.fori_loop(0, SEQ, pos_body, 0)

        start_gather(0, 0)
        start_gather(1, 1)

        def pair_body(g, carry):
            for dc in range(2):
                c = g * 2 + dc
                gb = dc
                wait_gather(gb)

                @pl.when(g > 0)
                def _():
                    wait_wb(gb)

                compute(gb, gb)

                @pl.when(c + 2 < n_chunks)
                def _():
                    start_gather(c + 2, gb)

                start_wb(c, gb)
            return carry

        lax.fori_loop(0, n_chunks // 2, pair_body, 0)
        for ob in range(2):
            wait_wb(ob)

    return body(idx_flat, wide, pe_flat)


def kernel(encoded_words, table, pe):
    batch, seq = encoded_words.shape
    vocab = table.shape[0]
    n_rows = batch * seq
    idx_flat = encoded_words.reshape(n_rows)
    pe_flat = pe.reshape(pe.shape[1] * pe.shape[2])[: seq * D_MODEL]
    wide = _widen_transpose(table.T, vocab)
    return _gather_pe(idx_flat, wide, pe_flat, batch, seq)


# speedup vs baseline: 1.3868x; 1.3868x over previous
import functools
import math

import jax
import jax.numpy as jnp
from jax import lax
from jax.experimental import pallas as pl
from jax.experimental.pallas import tpu as pltpu
from jax.experimental.pallas import tpu_sc as plsc

D_MODEL = 64
SEQ = 50
SCALE = math.sqrt(D_MODEL)

_info = plsc.get_sparse_core_info()
NC = _info.num_cores
NS = _info.num_subcores
NW = NC * NS

CHUNK_BATCH = 4
CHUNK_ROWS = CHUNK_BATCH * SEQ
SUBS = ((0, 128), (128, 72))
TBLK = 8192


def _widen_transpose(table_t, vocab):
    grid = (vocab + TBLK - 1) // TBLK

    def tbody(x_ref, o_ref):
        o_ref[:, 0:D_MODEL] = jnp.transpose(x_ref[...])

    return pl.pallas_call(
        tbody,
        grid=(grid,),
        in_specs=[pl.BlockSpec((D_MODEL, TBLK), lambda i: (0, i))],
        out_specs=pl.BlockSpec((TBLK, 128), lambda i: (i, 0)),
        out_shape=jax.ShapeDtypeStruct((vocab, 128), jnp.float32),
    )(table_t)


def _gather_pe(idx_flat, wide, pe_flat, batch, seq):
    n_rows = batch * seq
    rows_per_w = n_rows // NW
    n_chunks = rows_per_w // CHUNK_ROWS
    batches_per_w = rows_per_w // seq
    mesh = plsc.VectorSubcoreMesh(core_axis_name="c", subcore_axis_name="s")

    @functools.partial(
        pl.kernel,
        mesh=mesh,
        out_type=jax.ShapeDtypeStruct((batch, seq, D_MODEL), jnp.float32),
        scratch_types=[
            pltpu.VMEM((rows_per_w,), jnp.int32),
            [pltpu.VMEM((CHUNK_ROWS, 128), jnp.float32)] * 2,
            [pltpu.VMEM((CHUNK_BATCH, SEQ, D_MODEL), jnp.float32)] * 2,
            pltpu.VMEM((SEQ * D_MODEL,), jnp.float32),
            [pltpu.SemaphoreType.DMA] * 2,
            [pltpu.SemaphoreType.DMA] * 2,
        ],
        compiler_params=pltpu.CompilerParams(use_tc_tiling_on_sc=True),
    )
    def body(idx_hbm, wide_hbm, pe_hbm, out_hbm, idx_all, gbufs, obufs,
             pe_v, gsems, wsems):
        wid = lax.axis_index("s") * NC + lax.axis_index("c")
        wbase = pl.multiple_of(wid * rows_per_w, rows_per_w)
        wbatch = wid * batches_per_w
        pltpu.sync_copy(pe_hbm, pe_v)
        pltpu.sync_copy(idx_hbm.at[pl.ds(wbase, rows_per_w)], idx_all)

        def start_gather(c, gb):
            for off, sz in SUBS:
                pltpu.async_copy(
                    wide_hbm.at[
                        idx_all.at[
                            pl.ds(pl.multiple_of(c * CHUNK_ROWS + off, 8), sz)
                        ]
                    ],
                    gbufs[gb].at[pl.ds(off, sz)],
                    gsems[gb],
                )

        def wait_gather(gb):
            for off, sz in SUBS:
                pltpu.make_async_copy(
                    wide_hbm.at[idx_all.at[pl.ds(off, sz)]],
                    gbufs[gb].at[pl.ds(off, sz)],
                    gsems[gb],
                ).wait()

        def start_wb(c, ob):
            for bb in range(CHUNK_BATCH):
                pltpu.async_copy(
                    obufs[ob].at[bb],
                    out_hbm.at[wbatch + c * CHUNK_BATCH + bb],
                    wsems[ob],
                )

        def wait_wb(ob):
            for bb in range(CHUNK_BATCH):
                pltpu.make_async_copy(
                    obufs[ob].at[bb],
                    out_hbm.at[wbatch],
                    wsems[ob],
                ).wait()

        def compute(gb, ob):
            def pos_body(s, carry):
                for q in range(D_MODEL // 16):
                    peq = pe_v[pl.ds(pl.multiple_of(s * D_MODEL, 8) + q * 16,
                                     16)]
                    for bb in range(CHUNK_BATCH):
                        obufs[ob][bb, s, pl.ds(q * 16, 16)] = (
                            gbufs[gb][bb * SEQ + s, pl.ds(q * 16, 16)] * SCALE
                            + peq
                        )
                return carry

            lax.fori_loop(0, SEQ, pos_body, 0)

        start_gather(0, 0)
        start_gather(1, 1)

        def pair_body(g, carry):
            for dc in range(2):
                c = g * 2 + dc
                gb = dc
                wait_gather(gb)

                @pl.when(g > 0)
                def _():
                    wait_wb(gb)

                compute(gb, gb)

                @pl.when(c + 2 < n_chunks)
                def _():
                    start_gather(c + 2, gb)

                start_wb(c, gb)
            return carry

        lax.fori_loop(0, n_chunks // 2, pair_body, 0)
        for ob in range(2):
            wait_wb(ob)

    return body(idx_flat, wide, pe_flat)


def kernel(encoded_words, table, pe):
    batch, seq = encoded_words.shape
    vocab = table.shape[0]
    n_rows = batch * seq
    idx_flat = encoded_words.reshape(n_rows)
    pe_flat = pe.reshape(pe.shape[1] * pe.shape[2])[: seq * D_MODEL]
    wide = _widen_transpose(table.T, vocab)
    return _gather_pe(idx_flat, wide, pe_flat, batch, seq)


# speedup vs baseline: 1.4506x; 1.0460x over previous
import functools
import math

import jax
import jax.numpy as jnp
from jax import lax
from jax.experimental import pallas as pl
from jax.experimental.pallas import tpu as pltpu
from jax.experimental.pallas import tpu_sc as plsc

D_MODEL = 64
SEQ = 50
SCALE = math.sqrt(D_MODEL)

_info = plsc.get_sparse_core_info()
NC = _info.num_cores
NS = _info.num_subcores
NW = NC * NS

CHUNK_BATCH = 4
CHUNK_ROWS = CHUNK_BATCH * SEQ
SUBS = ((0, 128), (128, 72))
TBLK = 16384


def _widen_transpose(table_t, vocab):
    grid = (vocab + TBLK - 1) // TBLK

    def tbody(x_ref, o_ref):
        o_ref[:, 0:D_MODEL] = jnp.transpose(x_ref[...])

    return pl.pallas_call(
        tbody,
        grid=(grid,),
        in_specs=[pl.BlockSpec((D_MODEL, TBLK), lambda i: (0, i))],
        out_specs=pl.BlockSpec((TBLK, 128), lambda i: (i, 0)),
        out_shape=jax.ShapeDtypeStruct((vocab, 128), jnp.float32),
    )(table_t)


def _gather_pe(idx_flat, wide, pe_flat, batch, seq):
    n_rows = batch * seq
    rows_per_w = n_rows // NW
    n_chunks = rows_per_w // CHUNK_ROWS
    batches_per_w = rows_per_w // seq
    mesh = plsc.VectorSubcoreMesh(core_axis_name="c", subcore_axis_name="s")

    @functools.partial(
        pl.kernel,
        mesh=mesh,
        out_type=jax.ShapeDtypeStruct((batch, seq, D_MODEL), jnp.float32),
        scratch_types=[
            pltpu.VMEM((rows_per_w,), jnp.int32),
            [pltpu.VMEM((CHUNK_ROWS, 128), jnp.float32)] * 2,
            [pltpu.VMEM((CHUNK_BATCH, SEQ, D_MODEL), jnp.float32)] * 2,
            pltpu.VMEM((SEQ * D_MODEL,), jnp.float32),
            [pltpu.SemaphoreType.DMA] * 2,
            [pltpu.SemaphoreType.DMA] * 2,
        ],
        compiler_params=pltpu.CompilerParams(use_tc_tiling_on_sc=True),
    )
    def body(idx_hbm, wide_hbm, pe_hbm, out_hbm, idx_all, gbufs, obufs,
             pe_v, gsems, wsems):
        wid = lax.axis_index("s") * NC + lax.axis_index("c")
        wbase = pl.multiple_of(wid * rows_per_w, rows_per_w)
        wbatch = wid * batches_per_w
        pltpu.sync_copy(pe_hbm, pe_v)
        pltpu.sync_copy(idx_hbm.at[pl.ds(wbase, rows_per_w)], idx_all)

        def start_gather(c, gb):
            for off, sz in SUBS:
                pltpu.async_copy(
                    wide_hbm.at[
                        idx_all.at[
                            pl.ds(pl.multiple_of(c * CHUNK_ROWS + off, 8), sz)
                        ]
                    ],
                    gbufs[gb].at[pl.ds(off, sz)],
                    gsems[gb],
                )

        def wait_gather(gb):
            for off, sz in SUBS:
                pltpu.make_async_copy(
                    wide_hbm.at[idx_all.at[pl.ds(off, sz)]],
                    gbufs[gb].at[pl.ds(off, sz)],
                    gsems[gb],
                ).wait()

        def start_wb(c, ob):
            for bb in range(CHUNK_BATCH):
                pltpu.async_copy(
                    obufs[ob].at[bb],
                    out_hbm.at[wbatch + c * CHUNK_BATCH + bb],
                    wsems[ob],
                )

        def wait_wb(ob):
            for bb in range(CHUNK_BATCH):
                pltpu.make_async_copy(
                    obufs[ob].at[bb],
                    out_hbm.at[wbatch],
                    wsems[ob],
                ).wait()

        def compute(gb, ob):
            def pos_body(s, carry):
                for q in range(D_MODEL // 16):
                    peq = pe_v[pl.ds(pl.multiple_of(s * D_MODEL, 8) + q * 16,
                                     16)]
                    for bb in range(CHUNK_BATCH):
                        obufs[ob][bb, s, pl.ds(q * 16, 16)] = (
                            gbufs[gb][bb * SEQ + s, pl.ds(q * 16, 16)] * SCALE
                            + peq
                        )
                return carry

            lax.fori_loop(0, SEQ, pos_body, 0)

        start_gather(0, 0)
        start_gather(1, 1)

        def pair_body(g, carry):
            for dc in range(2):
                c = g * 2 + dc
                gb = dc
                wait_gather(gb)

                @pl.when(g > 0)
                def _():
                    wait_wb(gb)

                compute(gb, gb)

                @pl.when(c + 2 < n_chunks)
                def _():
                    start_gather(c + 2, gb)

                start_wb(c, gb)
            return carry

        lax.fori_loop(0, n_chunks // 2, pair_body, 0)
        for ob in range(2):
            wait_wb(ob)

    return body(idx_flat, wide, pe_flat)


def kernel(encoded_words, table, pe):
    batch, seq = encoded_words.shape
    vocab = table.shape[0]
    n_rows = batch * seq
    idx_flat = encoded_words.reshape(n_rows)
    pe_flat = pe.reshape(pe.shape[1] * pe.shape[2])[: seq * D_MODEL]
    wide = _widen_transpose(table.T, vocab)
    return _gather_pe(idx_flat, wide, pe_flat, batch, seq)


# speedup vs baseline: 1.4726x; 1.0152x over previous
import functools
import math

import jax
import jax.numpy as jnp
from jax import lax
from jax.experimental import pallas as pl
from jax.experimental.pallas import tpu as pltpu
from jax.experimental.pallas import tpu_sc as plsc

D_MODEL = 64
SEQ = 50
SCALE = math.sqrt(D_MODEL)

_info = plsc.get_sparse_core_info()
NC = _info.num_cores
NS = _info.num_subcores
NW = NC * NS

CHUNK_BATCH = 4
CHUNK_ROWS = CHUNK_BATCH * SEQ
SUBS = ((0, 128), (128, 72))
TBLK = 32768


def _widen_transpose(table_t, vocab):
    grid = (vocab + TBLK - 1) // TBLK

    def tbody(x_ref, o_ref):
        o_ref[:, 0:D_MODEL] = jnp.transpose(x_ref[...])

    return pl.pallas_call(
        tbody,
        grid=(grid,),
        in_specs=[pl.BlockSpec((D_MODEL, TBLK), lambda i: (0, i))],
        out_specs=pl.BlockSpec((TBLK, 128), lambda i: (i, 0)),
        out_shape=jax.ShapeDtypeStruct((vocab, 128), jnp.float32),
    )(table_t)


def _gather_pe(idx_flat, wide, pe_flat, batch, seq):
    n_rows = batch * seq
    rows_per_w = n_rows // NW
    n_chunks = rows_per_w // CHUNK_ROWS
    batches_per_w = rows_per_w // seq
    mesh = plsc.VectorSubcoreMesh(core_axis_name="c", subcore_axis_name="s")

    @functools.partial(
        pl.kernel,
        mesh=mesh,
        out_type=jax.ShapeDtypeStruct((batch, seq, D_MODEL), jnp.float32),
        scratch_types=[
            pltpu.VMEM((rows_per_w,), jnp.int32),
            [pltpu.VMEM((CHUNK_ROWS, 128), jnp.float32)] * 2,
            [pltpu.VMEM((CHUNK_BATCH, SEQ, D_MODEL), jnp.float32)] * 2,
            pltpu.VMEM((SEQ * D_MODEL,), jnp.float32),
            [pltpu.SemaphoreType.DMA] * 2,
            [pltpu.SemaphoreType.DMA] * 2,
        ],
        compiler_params=pltpu.CompilerParams(use_tc_tiling_on_sc=True),
    )
    def body(idx_hbm, wide_hbm, pe_hbm, out_hbm, idx_all, gbufs, obufs,
             pe_v, gsems, wsems):
        wid = lax.axis_index("s") * NC + lax.axis_index("c")
        wbase = pl.multiple_of(wid * rows_per_w, rows_per_w)
        wbatch = wid * batches_per_w
        pltpu.sync_copy(pe_hbm, pe_v)
        pltpu.sync_copy(idx_hbm.at[pl.ds(wbase, rows_per_w)], idx_all)

        def start_gather(c, gb):
            for off, sz in SUBS:
                pltpu.async_copy(
                    wide_hbm.at[
                        idx_all.at[
                            pl.ds(pl.multiple_of(c * CHUNK_ROWS + off, 8), sz)
                        ]
                    ],
                    gbufs[gb].at[pl.ds(off, sz)],
                    gsems[gb],
                )

        def wait_gather(gb):
            for off, sz in SUBS:
                pltpu.make_async_copy(
                    wide_hbm.at[idx_all.at[pl.ds(off, sz)]],
                    gbufs[gb].at[pl.ds(off, sz)],
                    gsems[gb],
                ).wait()

        def start_wb(c, ob):
            for bb in range(CHUNK_BATCH):
                pltpu.async_copy(
                    obufs[ob].at[bb],
                    out_hbm.at[wbatch + c * CHUNK_BATCH + bb],
                    wsems[ob],
                )

        def wait_wb(ob):
            for bb in range(CHUNK_BATCH):
                pltpu.make_async_copy(
                    obufs[ob].at[bb],
                    out_hbm.at[wbatch],
                    wsems[ob],
                ).wait()

        def compute(gb, ob):
            def pos_body(s, carry):
                for q in range(D_MODEL // 16):
                    peq = pe_v[pl.ds(pl.multiple_of(s * D_MODEL, 8) + q * 16,
                                     16)]
                    for bb in range(CHUNK_BATCH):
                        obufs[ob][bb, s, pl.ds(q * 16, 16)] = (
                            gbufs[gb][bb * SEQ + s, pl.ds(q * 16, 16)] * SCALE
                            + peq
                        )
                return carry

            lax.fori_loop(0, SEQ, pos_body, 0)

        start_gather(0, 0)
        start_gather(1, 1)

        def pair_body(g, carry):
            for dc in range(2):
                c = g * 2 + dc
                gb = dc
                wait_gather(gb)

                @pl.when(g > 0)
                def _():
                    wait_wb(gb)

                compute(gb, gb)

                @pl.when(c + 2 < n_chunks)
                def _():
                    start_gather(c + 2, gb)

                start_wb(c, gb)
            return carry

        lax.fori_loop(0, n_chunks // 2, pair_body, 0)
        for ob in range(2):
            wait_wb(ob)

    return body(idx_flat, wide, pe_flat)


def kernel(encoded_words, table, pe):
    batch, seq = encoded_words.shape
    vocab = table.shape[0]
    n_rows = batch * seq
    idx_flat = encoded_words.reshape(n_rows)
    pe_flat = pe.reshape(pe.shape[1] * pe.shape[2])[: seq * D_MODEL]
    wide = _widen_transpose(table.T, vocab)
    return _gather_pe(idx_flat, wide, pe_flat, batch, seq)
